# Initial kernel scaffold; baseline (speedup 1.0000x reference)
import jax
import jax.numpy as jnp
from jax.experimental import pallas as pl


def kernel(x, edge_index, W1_0, b1_0, W2_0, b2_0, W1_1, b1_1, W2_1, b2_1, W1_2, b1_2, W2_2, b2_2):
    raise NotImplementedError("write your pallas kernel here")


# speedup vs baseline: 6.2663x; 6.2663x over previous
import functools

import jax
import jax.numpy as jnp
from jax import lax
from jax.experimental import pallas as pl
from jax.experimental.pallas import tpu as pltpu
from jax.experimental.pallas import tpu_sc as plsc

N = 10000
E = 320000
NC = 2
NS = 16
NW = NC * NS
EDGES_PER_W = E // NW
CHUNK = 128
NFULL = EDGES_PER_W // CHUNK
TAIL = EDGES_PER_W - NFULL * CHUNK
ROWS_PER_TILE = 624
EXTRA_ROWS = N - NS * ROWS_PER_TILE


def _make_seg_sum(F):
    mesh = plsc.VectorSubcoreMesh(core_axis_name="c", subcore_axis_name="s")

    @functools.partial(
        pl.kernel,
        mesh=mesh,
        compiler_params=pltpu.CompilerParams(use_tc_tiling_on_sc=False),
        out_type=jax.ShapeDtypeStruct((NC * N, F), jnp.float32),
        scratch_types=[
            pltpu.VMEM((CHUNK,), jnp.int32),
            pltpu.VMEM((CHUNK,), jnp.int32),
            pltpu.VMEM((TAIL,), jnp.int32),
            pltpu.VMEM((TAIL,), jnp.int32),
            pltpu.VMEM((CHUNK, F), jnp.float32),
            pltpu.VMEM((TAIL, F), jnp.float32),
            pltpu.VMEM_SHARED((N, F), jnp.float32),
            pltpu.SemaphoreType.DMA,
        ],
    )
    def seg_sum(h_hbm, src_hbm, dst_hbm, out_hbm,
                idx_s, idx_d, idx_st, idx_dt, rows, rows_t, acc, sem):
        c = lax.axis_index("c")
        s = lax.axis_index("s")
        wid = s * NC + c
        ebase = wid * EDGES_PER_W
        rbase = s * ROWS_PER_TILE

        zeros16 = jnp.zeros((16,), jnp.float32)
        n_vec = CHUNK * F // 16

        def zbody(i, carry):
            rows[i // (F // 16), pl.ds((i % (F // 16)) * 16, 16)] = zeros16
            return carry

        lax.fori_loop(0, n_vec, zbody, 0)
        nfullcopy = ROWS_PER_TILE // CHUNK
        for j in range(nfullcopy):
            pltpu.sync_copy(rows, acc.at[pl.ds(rbase + j * CHUNK, CHUNK)])
        rem = ROWS_PER_TILE - nfullcopy * CHUNK
        if rem:
            pltpu.sync_copy(rows.at[pl.ds(0, rem)],
                            acc.at[pl.ds(rbase + nfullcopy * CHUNK, rem)])

        @pl.when(s == NS - 1)
        def _zero_extra():
            pltpu.sync_copy(rows.at[pl.ds(0, EXTRA_ROWS)],
                            acc.at[pl.ds(NS * ROWS_PER_TILE, EXTRA_ROWS)])

        plsc.subcore_barrier()

        def body(i, carry):
            base = ebase + i * CHUNK
            pltpu.sync_copy(src_hbm.at[pl.ds(base, CHUNK)], idx_s)
            pltpu.sync_copy(dst_hbm.at[pl.ds(base, CHUNK)], idx_d)
            pltpu.async_copy(h_hbm.at[idx_s], rows, sem).wait()
            pltpu.sync_copy(rows, acc.at[idx_d], add=True)
            return carry

        lax.fori_loop(0, NFULL, body, 0)
        if TAIL:
            tbase = ebase + NFULL * CHUNK
            pltpu.sync_copy(src_hbm.at[pl.ds(tbase, TAIL)], idx_st)
            pltpu.sync_copy(dst_hbm.at[pl.ds(tbase, TAIL)], idx_dt)
            pltpu.async_copy(h_hbm.at[idx_st], rows_t, sem).wait()
            pltpu.sync_copy(rows_t, acc.at[idx_dt], add=True)
        plsc.subcore_barrier()

        pltpu.sync_copy(acc.at[pl.ds(rbase, ROWS_PER_TILE)],
                        out_hbm.at[pl.ds(c * N + rbase, ROWS_PER_TILE)])

        @pl.when(s == NS - 1)
        def _write_extra():
            pltpu.sync_copy(
                acc.at[pl.ds(NS * ROWS_PER_TILE, EXTRA_ROWS)],
                out_hbm.at[pl.ds(c * N + NS * ROWS_PER_TILE, EXTRA_ROWS)])

    return seg_sum


def _make_mlp(Fin, R=1000):
    H = 64
    grid = (N // R,)

    def body(h_ref, a0_ref, a1_ref, w1_ref, b1_ref, w2_ref, b2_ref, o_ref):
        u = h_ref[...] + a0_ref[...] + a1_ref[...]
        z = jnp.dot(u, w1_ref[...], preferred_element_type=jnp.float32)
        z = jnp.maximum(z + b1_ref[...], 0.0)
        o = jnp.dot(z, w2_ref[...], preferred_element_type=jnp.float32)
        o_ref[...] = jnp.maximum(o + b2_ref[...], 0.0)

    return pl.pallas_call(
        body,
        grid=grid,
        in_specs=[
            pl.BlockSpec((R, Fin), lambda i: (i, 0)),
            pl.BlockSpec((R, Fin), lambda i: (i, 0)),
            pl.BlockSpec((R, Fin), lambda i: (i + N // R, 0)),
            pl.BlockSpec((Fin, H), lambda i: (0, 0)),
            pl.BlockSpec((1, H), lambda i: (0, 0)),
            pl.BlockSpec((H, H), lambda i: (0, 0)),
            pl.BlockSpec((1, H), lambda i: (0, 0)),
        ],
        out_specs=pl.BlockSpec((R, H), lambda i: (i, 0)),
        out_shape=jax.ShapeDtypeStruct((N, H), jnp.float32),
    )


def kernel(x, edge_index, W1_0, b1_0, W2_0, b2_0, W1_1, b1_1, W2_1, b2_1,
           W1_2, b1_2, W2_2, b2_2):
    src = edge_index[0]
    dst = edge_index[1]
    params = [(W1_0, b1_0, W2_0, b2_0), (W1_1, b1_1, W2_1, b2_1),
              (W1_2, b1_2, W2_2, b2_2)]
    h = x
    outs = []
    for (W1, b1, W2, b2) in params:
        F = h.shape[1]
        agg2 = _make_seg_sum(F)(h, src, dst)
        h = _make_mlp(F)(h, agg2, agg2, W1, b1.reshape(1, -1),
                         W2, b2.reshape(1, -1))
        outs.append(h)
    return jnp.concatenate(outs, axis=1)


# speedup vs baseline: 12.3098x; 1.9645x over previous
import functools

import jax
import jax.numpy as jnp
from jax import lax
from jax.experimental import pallas as pl
from jax.experimental.pallas import tpu as pltpu
from jax.experimental.pallas import tpu_sc as plsc

N = 10000
E = 320000
NC = 2
NS = 16
NW = NC * NS
CHUNK = 128
NCHUNKS = E // CHUNK
CH_PER_W = NCHUNKS // NW
EXTRA_CHUNKS = NCHUNKS - CH_PER_W * NW
ROWS_PER_TILE = 624
EXTRA_ROWS = N - NS * ROWS_PER_TILE


def _make_seg_sum(F):
    mesh = plsc.VectorSubcoreMesh(core_axis_name="c", subcore_axis_name="s")

    @functools.partial(
        pl.kernel,
        mesh=mesh,
        compiler_params=pltpu.CompilerParams(use_tc_tiling_on_sc=False),
        out_type=jax.ShapeDtypeStruct((NC * N, F), jnp.float32),
        scratch_types=[
            pltpu.VMEM((CH_PER_W * CHUNK,), jnp.int32),
            pltpu.VMEM((2, CHUNK), jnp.int32),
            pltpu.VMEM((CHUNK, F), jnp.float32),
            pltpu.VMEM((CHUNK, F), jnp.float32),
            pltpu.VMEM_SHARED((N, F), jnp.float32),
            pltpu.SemaphoreType.DMA,
            pltpu.SemaphoreType.DMA,
            pltpu.SemaphoreType.DMA,
            pltpu.SemaphoreType.DMA,
        ],
    )
    def seg_sum(h_hbm, src_hbm, dst2d_hbm, out_hbm,
                idx_s, idx_d, rows0, rows1, acc,
                semg0, semg1, semd0, semd1):
        c = lax.axis_index("c")
        s = lax.axis_index("s")
        wid = s * NC + c
        cbase = wid * CH_PER_W
        rbase = s * ROWS_PER_TILE

        zeros16 = jnp.zeros((16,), jnp.float32)
        fvec = F // 16

        @pl.loop(0, CHUNK * fvec, unroll=8)
        def _zero(i):
            rows0[i // fvec, pl.ds((i % fvec) * 16, 16)] = zeros16

        nfullcopy = ROWS_PER_TILE // CHUNK
        for j in range(nfullcopy):
            pltpu.sync_copy(rows0, acc.at[pl.ds(rbase + j * CHUNK, CHUNK)])
        rem = ROWS_PER_TILE - nfullcopy * CHUNK
        if rem:
            pltpu.sync_copy(rows0.at[pl.ds(0, rem)],
                            acc.at[pl.ds(rbase + nfullcopy * CHUNK, rem)])

        @pl.when(s == NS - 1)
        def _zero_extra():
            pltpu.sync_copy(rows0.at[pl.ds(0, EXTRA_ROWS)],
                            acc.at[pl.ds(NS * ROWS_PER_TILE, EXTRA_ROWS)])

        pltpu.sync_copy(src_hbm.at[pl.ds(cbase * CHUNK, CH_PER_W * CHUNK)],
                        idx_s)

        rows = (rows0, rows1)
        semg = (semg0, semg1)
        semd = (semd0, semd1)
        pltpu.async_copy(h_hbm.at[idx_s.at[pl.ds(0, CHUNK)]], rows0, semg0)
        pltpu.async_copy(dst2d_hbm.at[pl.ds(cbase, 1)],
                         idx_d.at[pl.ds(0, 1)], semd0)
        plsc.subcore_barrier()

        @pl.loop(0, CH_PER_W, step=2)
        def _go(i):
            for b in range(2):
                cc = i + b
                nxt = 1 - b

                def _issue(nc=cc + 1, nb=nxt):
                    pltpu.async_copy(
                        h_hbm.at[idx_s.at[pl.ds(nc * CHUNK, CHUNK)]],
                        rows[nb], semg[nb])
                    pltpu.async_copy(dst2d_hbm.at[pl.ds(cbase + nc, 1)],
                                     idx_d.at[pl.ds(nb, 1)], semd[nb])

                if b == 0:
                    _issue()
                else:
                    pl.when(i < CH_PER_W - 2)(_issue)
                pltpu.make_async_copy(h_hbm.at[idx_s.at[pl.ds(0, CHUNK)]],
                                      rows[b], semg[b]).wait()
                pltpu.make_async_copy(dst2d_hbm.at[pl.ds(0, 1)],
                                      idx_d.at[pl.ds(b, 1)], semd[b]).wait()
                pltpu.sync_copy(rows[b], acc.at[idx_d.at[b]], add=True)

        @pl.when(wid < EXTRA_CHUNKS)
        def _extra():
            ck = NW * CH_PER_W + wid
            pltpu.sync_copy(src_hbm.at[pl.ds(ck * CHUNK, CHUNK)],
                            idx_s.at[pl.ds(0, CHUNK)])
            pltpu.sync_copy(dst2d_hbm.at[pl.ds(ck, 1)], idx_d.at[pl.ds(0, 1)])
            pltpu.async_copy(h_hbm.at[idx_s.at[pl.ds(0, CHUNK)]], rows0,
                             semg0).wait()
            pltpu.sync_copy(rows0, acc.at[idx_d.at[0]], add=True)

        plsc.subcore_barrier()

        pltpu.sync_copy(acc.at[pl.ds(rbase, ROWS_PER_TILE)],
                        out_hbm.at[pl.ds(c * N + rbase, ROWS_PER_TILE)])

        @pl.when(s == NS - 1)
        def _write_extra():
            pltpu.sync_copy(
                acc.at[pl.ds(NS * ROWS_PER_TILE, EXTRA_ROWS)],
                out_hbm.at[pl.ds(c * N + NS * ROWS_PER_TILE, EXTRA_ROWS)])

    return seg_sum


def _make_mlp(Fin, R=1000):
    H = 64
    grid = (N // R,)

    def body(h_ref, a0_ref, a1_ref, w1_ref, b1_ref, w2_ref, b2_ref, o_ref):
        u = h_ref[...] + a0_ref[...] + a1_ref[...]
        z = jnp.dot(u, w1_ref[...], preferred_element_type=jnp.float32)
        z = jnp.maximum(z + b1_ref[...], 0.0)
        o = jnp.dot(z, w2_ref[...], preferred_element_type=jnp.float32)
        o_ref[...] = jnp.maximum(o + b2_ref[...], 0.0)

    return pl.pallas_call(
        body,
        grid=grid,
        in_specs=[
            pl.BlockSpec((R, Fin), lambda i: (i, 0)),
            pl.BlockSpec((R, Fin), lambda i: (i, 0)),
            pl.BlockSpec((R, Fin), lambda i: (i + N // R, 0)),
            pl.BlockSpec((Fin, H), lambda i: (0, 0)),
            pl.BlockSpec((1, H), lambda i: (0, 0)),
            pl.BlockSpec((H, H), lambda i: (0, 0)),
            pl.BlockSpec((1, H), lambda i: (0, 0)),
        ],
        out_specs=pl.BlockSpec((R, H), lambda i: (i, 0)),
        out_shape=jax.ShapeDtypeStruct((N, H), jnp.float32),
    )


def kernel(x, edge_index, W1_0, b1_0, W2_0, b2_0, W1_1, b1_1, W2_1, b2_1,
           W1_2, b1_2, W2_2, b2_2):
    src = edge_index[0]
    dst2d = edge_index[1].reshape(NCHUNKS, CHUNK)
    params = [(W1_0, b1_0, W2_0, b2_0), (W1_1, b1_1, W2_1, b2_1),
              (W1_2, b1_2, W2_2, b2_2)]
    h = x
    outs = []
    for (W1, b1, W2, b2) in params:
        F = h.shape[1]
        agg2 = _make_seg_sum(F)(h, src, dst2d)
        h = _make_mlp(F)(h, agg2, agg2, W1, b1.reshape(1, -1),
                         W2, b2.reshape(1, -1))
        outs.append(h)
    return jnp.concatenate(outs, axis=1)
